# baseline (speedup 1.0000x reference)
import jax
import jax.numpy as jnp
from jax import lax
from jax.experimental import pallas as pl
from jax.experimental.pallas import tpu as pltpu
from jax.experimental.pallas import tpu_sc as plsc

NUM_EMB = 1000
D = 64
B = 4096
L = 200

_INFO = plsc.get_sparse_core_info()
NC = _INFO.num_cores
NS = _INFO.num_subcores
NW = NC * NS
ROWS_PER_W = B // NW
CHUNK = ROWS_PER_W * L
TROWS = 208
PSTR = 129
NSLOT = 4
NRND = L // NSLOT


def _body(cm_hbm, w_hbm, out_hbm, cm_v, pos_t, tab_v, stage_v, *sems):
    wid = lax.axis_index("s") * NC + lax.axis_index("c")
    base = wid * CHUNK

    pltpu.sync_copy(w_hbm.at[pl.ds(0, TROWS)], tab_v)
    pltpu.sync_copy(cm_hbm.at[pl.ds(base, CHUNK)], cm_v.at[pl.ds(0, CHUNK)])

    iota = lax.iota(jnp.int32, 16)
    lane_lt8 = iota < jnp.full((16,), 8, jnp.int32)
    ones = jnp.full((16,), 1, jnp.int32)
    n_chunks = (L + 15) // 16

    def row_body(r, _):
        carry = jnp.full((16,), 0, jnp.int32)
        row0 = r * L
        for j in range(n_chunks):
            off = row0 + j * 16
            last = j == n_chunks - 1
            x = cm_v[pl.ds(off, 16)]
            m = (x == ones).astype(jnp.int32)
            if last:
                m = m * lane_lt8.astype(jnp.int32)
            c = plsc.cumsum(m)
            pos = (c + carry) * m
            carry = carry + jnp.full((16,), jnp.sum(m), jnp.int32)
            addr = (iota + j * 16) * PSTR + r
            if last:
                plsc.store_scatter(pos_t, [addr], pos, mask=lane_lt8)
            else:
                plsc.store_scatter(pos_t, [addr], pos)
        return 0

    lax.fori_loop(0, ROWS_PER_W, row_body, 0)

    passes = []
    for m in range(16):
        for n in range(4):
            c = ((iota + m) & 15) + 16 * n
            passes.append((c, c >> 3, (c & 7) * 128))

    def fill(l, s):
        def k_body(k, _):
            pos = pos_t[pl.ds(l * PSTR + k * 16, 16)]
            blane = iota + k * 16
            for c, c_hi, c_lo128 in passes:
                vals = plsc.load_gather(tab_v, [pos, c])
                plsc.store_scatter(stage_v.at[s], [c_hi, c_lo128 + blane], vals)
            return 0

        lax.fori_loop(0, ROWS_PER_W // 16, k_body, 0)

    def dst(l):
        return out_hbm.at[l, :, wid]

    for s in range(NSLOT):
        fill(s, s)
        pltpu.async_copy(stage_v.at[s], dst(s), sems[s])

    def round_body(rd, _):
        for s in range(NSLOT):
            l = rd * NSLOT + s
            pltpu.make_async_copy(
                stage_v.at[s], dst(l - NSLOT), sems[s]
            ).wait()
            fill(l, s)
            pltpu.async_copy(stage_v.at[s], dst(l), sems[s])
        return 0

    lax.fori_loop(1, NRND, round_body, 0)

    for s in range(NSLOT):
        pltpu.make_async_copy(
            stage_v.at[s], dst(L - NSLOT + s), sems[s]
        ).wait()


def _kernel_impl(chain_mask, weight):
    cm1d = chain_mask.reshape(B * L)
    run = pl.kernel(
        _body,
        out_type=jax.ShapeDtypeStruct((L, D // 8, B // 128, 8 * 128),
                                      jnp.float32),
        mesh=plsc.VectorSubcoreMesh(core_axis_name="c", subcore_axis_name="s"),
        compiler_params=pltpu.CompilerParams(
            use_tc_tiling_on_sc=False, needs_layout_passes=False
        ),
        scratch_types=[
            pltpu.VMEM((CHUNK + 16,), jnp.int32),
            pltpu.VMEM((L * PSTR + 16,), jnp.int32),
            pltpu.VMEM((TROWS, D), jnp.float32),
            pltpu.VMEM((NSLOT, D // 8, 8 * 128), jnp.float32),
        ] + [pltpu.SemaphoreType.DMA] * NSLOT,
    )
    out = run(cm1d, weight).reshape(L, D // 8, B // 128, 8, 128)
    return out.transpose(2, 4, 0, 1, 3).reshape(B, L, D)


kernel = jax.jit(_kernel_impl)
